# baseline (speedup 1.0000x reference)
import functools

import jax
import jax.numpy as jnp
from jax.experimental import pallas as pl


def _enc_conv_kernel(x_ref, w_ref, o_ref, *, Ho, Wo, relu):
    C = x_ref.shape[3]
    acc = None
    for t in range(4):
        dy, dx = t // 2, t % 2
        xs = x_ref[0, dy:dy + Ho, dx:dx + Wo, :].reshape(Ho * Wo, C)
        p = jax.lax.dot_general(xs, w_ref[t], (((1,), (0,)), ((), ())),
                                preferred_element_type=jnp.float32)
        acc = p if acc is None else acc + p
    if relu:
        acc = jnp.maximum(acc, 0.0)
    o_ref[0] = acc.reshape(Ho, Wo, o_ref.shape[3])


def _enc_conv(x_s2d, taps, Ho, Wo, relu):
    B, Hu, Wu, C = x_s2d.shape
    Co = taps.shape[2]
    kern = functools.partial(_enc_conv_kernel, Ho=Ho, Wo=Wo, relu=relu)
    return pl.pallas_call(
        kern,
        grid=(B,),
        in_specs=[
            pl.BlockSpec((1, Hu, Wu, C), lambda b: (b, 0, 0, 0)),
            pl.BlockSpec((4, C, Co), lambda b: (0, 0, 0)),
        ],
        out_specs=pl.BlockSpec((1, Ho, Wo, Co), lambda b: (b, 0, 0, 0)),
        out_shape=jax.ShapeDtypeStruct((B, Ho, Wo, Co), jnp.float32),
    )(x_s2d, taps)


def _dec_conv_kernel(x_ref, w_ref, o_ref, *, Ho, Wo, relu):
    Ci = x_ref.shape[3]
    Co = o_ref.shape[4]
    for p in range(4):
        py, px = p // 2, p % 2
        acc = None
        for t in range(4):
            dy, dx = t // 2, t % 2
            xs = x_ref[0, py + dy:py + dy + Ho, px + dx:px + dx + Wo, :]
            xs = xs.reshape(Ho * Wo, Ci)
            q = jax.lax.dot_general(xs, w_ref[p, t], (((1,), (0,)), ((), ())),
                                    preferred_element_type=jnp.float32)
            acc = q if acc is None else acc + q
        if relu:
            acc = jnp.maximum(acc, 0.0)
        o_ref[0, p] = acc.reshape(Ho, Wo, Co)


def _dec_conv(xp, taps, Ho, Wo, relu):
    B, Hp, Wp, Ci = xp.shape
    Co = taps.shape[3]
    kern = functools.partial(_dec_conv_kernel, Ho=Ho, Wo=Wo, relu=relu)
    return pl.pallas_call(
        kern,
        grid=(B,),
        in_specs=[
            pl.BlockSpec((1, Hp, Wp, Ci), lambda b: (b, 0, 0, 0)),
            pl.BlockSpec((4, 4, Ci, Co), lambda b: (0, 0, 0, 0)),
        ],
        out_specs=pl.BlockSpec((1, 4, Ho, Wo, Co), lambda b: (b, 0, 0, 0, 0)),
        out_shape=jax.ShapeDtypeStruct((B, 4, Ho, Wo, Co), jnp.float32),
    )(xp, taps)


def _dec_conv9_kernel(x_ref, w_ref, o_ref, *, Ho, Wo, relu):
    Ci = x_ref.shape[3]
    acc = None
    for s in range(9):
        sy, sx = s // 3, s % 3
        xs = x_ref[0, sy:sy + Ho, sx:sx + Wo, :].reshape(Ho * Wo, Ci)
        q = jax.lax.dot_general(xs, w_ref[s], (((1,), (0,)), ((), ())),
                                preferred_element_type=jnp.float32)
        acc = q if acc is None else acc + q
    if relu:
        acc = jnp.maximum(acc, 0.0)
    o_ref[0] = acc.reshape(Ho, Wo, o_ref.shape[3])


def _dec_conv9(xp, taps9, Ho, Wo, relu):
    B, Hp, Wp, Ci = xp.shape
    Co4 = taps9.shape[2]
    kern = functools.partial(_dec_conv9_kernel, Ho=Ho, Wo=Wo, relu=relu)
    return pl.pallas_call(
        kern,
        grid=(B,),
        in_specs=[
            pl.BlockSpec((1, Hp, Wp, Ci), lambda b: (b, 0, 0, 0)),
            pl.BlockSpec((9, Ci, Co4), lambda b: (0, 0, 0)),
        ],
        out_specs=pl.BlockSpec((1, Ho, Wo, Co4), lambda b: (b, 0, 0, 0)),
        out_shape=jax.ShapeDtypeStruct((B, Ho, Wo, Co4), jnp.float32),
    )(xp, taps9)


def _vq_kernel(z_ref, cb_ref, cbt_ref, zq_ref, loss_ref):
    R = z_ref.shape[0]
    K = cb_ref.shape[0]
    z = z_ref[...]
    cbt = cbt_ref[...]
    cn = jnp.sum(cbt * cbt, axis=0, keepdims=True)
    zc = jax.lax.dot_general(z, cbt, (((1,), (0,)), ((), ())),
                             preferred_element_type=jnp.float32)
    d = cn - 2.0 * zc
    m = jnp.min(d, axis=1, keepdims=True)
    iota = jax.lax.broadcasted_iota(jnp.int32, (R, K), 1)
    idx = jnp.min(jnp.where(d == m, iota, K), axis=1, keepdims=True)
    oh = (iota == idx).astype(jnp.float32)
    zq = jax.lax.dot_general(oh, cb_ref[...], (((1,), (0,)), ((), ())),
                             preferred_element_type=jnp.float32)
    zq_ref[...] = zq
    diff = zq - z
    part = jnp.sum(diff * diff).reshape(1, 1)

    @pl.when(pl.program_id(0) == 0)
    def _():
        loss_ref[...] = part

    @pl.when(pl.program_id(0) != 0)
    def _():
        loss_ref[...] = loss_ref[...] + part


def _vq(z_flat, codebook):
    N, D = z_flat.shape
    K = codebook.shape[0]
    R = 784
    cbt = codebook.T
    zq, loss = pl.pallas_call(
        _vq_kernel,
        grid=(N // R,),
        in_specs=[
            pl.BlockSpec((R, D), lambda i: (i, 0)),
            pl.BlockSpec((K, D), lambda i: (0, 0)),
            pl.BlockSpec((D, K), lambda i: (0, 0)),
        ],
        out_specs=[
            pl.BlockSpec((R, D), lambda i: (i, 0)),
            pl.BlockSpec((1, 1), lambda i: (0, 0)),
        ],
        out_shape=[
            jax.ShapeDtypeStruct((N, D), jnp.float32),
            jax.ShapeDtypeStruct((1, 1), jnp.float32),
        ],
    )(z_flat, codebook, cbt)
    return zq, loss[0, 0]


def _enc_taps(W):
    Wt = jnp.transpose(W, (2, 3, 1, 0))
    Ci, Co = Wt.shape[2], Wt.shape[3]
    taps = [Wt[2 * dy:2 * dy + 2, 2 * dx:2 * dx + 2].reshape(4 * Ci, Co)
            for dy in range(2) for dx in range(2)]
    return jnp.stack(taps)


def _dec_taps(W):
    Wt = jnp.transpose(W, (2, 3, 1, 0))
    rows = []
    for py in range(2):
        for px in range(2):
            rows.append(jnp.stack([Wt[2 * dy + py, 2 * dx + px]
                                   for dy in range(2) for dx in range(2)]))
    return jnp.stack(rows)


def _dec_taps9(W):
    Wt = jnp.transpose(W, (2, 3, 1, 0))
    Ci, Co = Wt.shape[2], Wt.shape[3]
    mats = []
    for sy in range(3):
        for sx in range(3):
            blocks = []
            for py in range(2):
                for px in range(2):
                    dy, dx = sy - py, sx - px
                    if 0 <= dy <= 1 and 0 <= dx <= 1:
                        blocks.append(Wt[2 * dy + py, 2 * dx + px])
                    else:
                        blocks.append(jnp.zeros((Ci, Co), jnp.float32))
            mats.append(jnp.concatenate(blocks, axis=1))
    return jnp.stack(mats)


def _d2s_flat(o, Wo):
    B, H, W, C4 = o.shape
    Co = C4 // 4
    o = o[:, :, :Wo, :].reshape(B, H, Wo, 2, 2, Co)
    o = jnp.transpose(o, (0, 1, 3, 2, 4, 5))
    return o.reshape(B, 2 * H, 2 * Wo, Co)


def _s2d(x_nhwc):
    B, H, W, C = x_nhwc.shape
    xp = jnp.pad(x_nhwc, ((0, 0), (1, 1), (1, 1), (0, 0)))
    xp = xp.reshape(B, (H + 2) // 2, 2, (W + 2) // 2, 2, C)
    xp = jnp.transpose(xp, (0, 1, 3, 2, 4, 5))
    return xp.reshape(B, (H + 2) // 2, (W + 2) // 2, 4 * C)


def _pad_dec(x_nhwc, Wp):
    B, H, W, C = x_nhwc.shape
    return jnp.pad(x_nhwc, ((0, 0), (1, 1), (1, Wp - W - 1), (0, 0)))


def _d2s(o, Wo):
    B, _, H, W, Co = o.shape
    o = o[:, :, :, :Wo, :].reshape(B, 2, 2, H, Wo, Co)
    o = jnp.transpose(o, (0, 3, 1, 4, 2, 5))
    return o.reshape(B, 2 * H, 2 * Wo, Co)


def kernel(x, We1, We2, We3, Wd1, Wd2, Wd3, codebook):
    B = x.shape[0]

    x_nhwc = jnp.transpose(x, (0, 2, 3, 1))
    h = _enc_conv(_s2d(x_nhwc), _enc_taps(We1), 112, 112, True)
    h = _enc_conv(_s2d(h), _enc_taps(We2), 56, 56, True)
    z3in = _s2d(h)
    z3in = jnp.pad(z3in, ((0, 0), (0, 0), (0, 11), (0, 0)))
    zw = _enc_conv(z3in, _enc_taps(We3), 28, 32, False)
    z = zw[:, :, :28, :]

    z_flat = z.reshape(B * 28 * 28, 64)
    zq_flat, loss_sum = _vq(z_flat, codebook)
    vq_loss = 1.25 * loss_sum / (B * 64 * 28 * 28)
    zq = zq_flat.reshape(B, 28, 28, 64)

    g = _dec_conv(_pad_dec(zq, 40), _dec_taps(Wd1), 28, 32, True)
    g = _d2s(g, 28)
    g = _dec_conv(_pad_dec(g, 64), _dec_taps(Wd2), 56, 56, True)
    g = _d2s(g, 56)
    g = _dec_conv9(_pad_dec(g, 120), _dec_taps9(Wd3), 112, 112, False)
    decoded_nhwc = _d2s_flat(g, 112)

    decoded = jnp.transpose(decoded_nhwc, (0, 3, 1, 2))
    z_quantized = jnp.transpose(zq, (0, 3, 1, 2))
    return (decoded, z_quantized, vq_loss)
